# baseline (speedup 1.0000x reference)
import functools

import jax
import jax.numpy as jnp
from jax import lax
from jax.experimental import pallas as pl
from jax.experimental.pallas import tpu as pltpu
from jax.experimental.pallas import tpu_sc as plsc

N, E, D, NSUB = 10000, 320000, 128, 500
NP = 10240
SBINS = 512
NW = 32
C = 128
NCH = 80
EW = NCH * C
EP = NW * EW
RPS = NP // 16
SW = NP // NW
SCH = 64
NSC = SW // SCH
RB = 1024
GB = NP // RB

f32 = jnp.float32
i32 = jnp.int32
_Z = lambda: jnp.int32(0)

_mesh = plsc.VectorSubcoreMesh(core_axis_name="c", subcore_axis_name="s")


@functools.partial(
    pl.kernel,
    out_type=(jax.ShapeDtypeStruct((2, NP), f32),
              jax.ShapeDtypeStruct((2, NP), f32),
              jax.ShapeDtypeStruct((2, SBINS), f32)),
    mesh=_mesh,
    scratch_types=[pltpu.VMEM((NCH, C), i32), pltpu.VMEM((NCH, C), i32),
                   pltpu.VMEM((NSC, SCH), i32),
                   pltpu.VMEM((C,), f32), pltpu.VMEM((SCH,), f32),
                   pltpu.VMEM_SHARED((NP,), f32),
                   pltpu.VMEM_SHARED((NP,), f32),
                   pltpu.VMEM_SHARED((SBINS,), f32),
                   pltpu.SemaphoreType.DMA, pltpu.SemaphoreType.DMA,
                   pltpu.SemaphoreType.DMA, pltpu.SemaphoreType.DMA],
)
def _hist_sc(src_h, dst_h, sid_h, zflat_h, dego_h, degi_h, pres_h,
             sidx_all, didx_all, tidx_all, ones_c, ones_s,
             dego_sh, degi_sh, pres_sh, sA, sB, sC, sD):
    c = lax.axis_index("c")
    s = lax.axis_index("s")
    w = s * 2 + c
    pltpu.sync_copy(zflat_h.at[pl.ds(0, RPS)], dego_sh.at[pl.ds(s * RPS, RPS)])
    pltpu.sync_copy(zflat_h.at[pl.ds(0, RPS)], degi_sh.at[pl.ds(s * RPS, RPS)])

    @pl.when(s == 0)
    def _():
        pltpu.sync_copy(zflat_h.at[pl.ds(0, SBINS)], pres_sh)

    for k in range(C // 16):
        ones_c[pl.ds(k * 16, 16)] = jnp.ones((16,), f32)
    for k in range(SCH // 16):
        ones_s[pl.ds(k * 16, 16)] = jnp.ones((16,), f32)
    pltpu.sync_copy(src_h.at[w], sidx_all)
    pltpu.sync_copy(dst_h.at[w], didx_all)
    pltpu.sync_copy(sid_h.at[w], tidx_all)
    plsc.subcore_barrier()

    def ebody(j, carry):
        i0 = j * 2
        d0 = pltpu.async_copy(ones_c, dego_sh.at[sidx_all.at[i0]], sA, add=True)
        d1 = pltpu.async_copy(ones_c, degi_sh.at[didx_all.at[i0]], sB, add=True)
        d2 = pltpu.async_copy(ones_c, dego_sh.at[sidx_all.at[i0 + 1]], sC, add=True)
        d3 = pltpu.async_copy(ones_c, degi_sh.at[didx_all.at[i0 + 1]], sD, add=True)
        d0.wait()
        d1.wait()
        d2.wait()
        d3.wait()
        return carry

    lax.fori_loop(jnp.int32(0), jnp.int32(NCH // 2), ebody, jnp.int32(0))

    def sbody(j, carry):
        pltpu.sync_copy(ones_s, pres_sh.at[tidx_all.at[j]], add=True)
        return carry

    lax.fori_loop(jnp.int32(0), jnp.int32(NSC), sbody, jnp.int32(0))
    plsc.subcore_barrier()
    pltpu.sync_copy(dego_sh.at[pl.ds(s * RPS, RPS)],
                    dego_h.at[c, pl.ds(s * RPS, RPS)])
    pltpu.sync_copy(degi_sh.at[pl.ds(s * RPS, RPS)],
                    degi_h.at[c, pl.ds(s * RPS, RPS)])

    @pl.when(s == 0)
    def _():
        pltpu.sync_copy(pres_sh, pres_h.at[c])


@functools.partial(
    pl.kernel,
    out_type=jax.ShapeDtypeStruct((2, NP, D), f32),
    mesh=_mesh,
    scratch_types=[pltpu.VMEM((C,), i32), pltpu.VMEM((C,), i32),
                   pltpu.VMEM((C,), i32), pltpu.VMEM((C,), i32),
                   pltpu.VMEM((C, D), f32), pltpu.VMEM((C, D), f32),
                   pltpu.VMEM_SHARED((NP, D), f32),
                   pltpu.SemaphoreType.DMA, pltpu.SemaphoreType.DMA,
                   pltpu.SemaphoreType.DMA, pltpu.SemaphoreType.DMA,
                   pltpu.SemaphoreType.DMA],
)
def _spmm_sc(y_h, src_h, dst_h, zrows_h, part_h,
             si0, si1, di0, di1, rows0, rows1, acc_sh, g0, g1, s0, s1, zs):
    c = lax.axis_index("c")
    s = lax.axis_index("s")
    w = s * 2 + c
    for k in range(RPS // 64):
        pltpu.async_copy(zrows_h.at[pl.ds(0, 64)],
                         acc_sh.at[pl.ds(s * RPS + k * 64, 64)], zs)
    for k in range(RPS // 64):
        pltpu.make_async_copy(zrows_h.at[pl.ds(0, 64)],
                              acc_sh.at[pl.ds(s * RPS + k * 64, 64)], zs).wait()
    plsc.subcore_barrier()
    base0 = w * EW
    pltpu.sync_copy(src_h.at[pl.ds(base0, C)], si0)
    pltpu.sync_copy(dst_h.at[pl.ds(base0, C)], di0)
    pltpu.async_copy(y_h.at[si0], rows0, g0)
    pltpu.sync_copy(src_h.at[pl.ds(base0 + C, C)], si1)
    pltpu.sync_copy(dst_h.at[pl.ds(base0 + C, C)], di1)
    pltpu.async_copy(y_h.at[si1], rows1, g1)

    def jbody(j, carry):
        i0 = j * 2
        pltpu.make_async_copy(y_h.at[si0], rows0, g0).wait()
        pltpu.async_copy(rows0, acc_sh.at[di0], s0, add=True)
        pltpu.make_async_copy(y_h.at[si1], rows1, g1).wait()
        pltpu.async_copy(rows1, acc_sh.at[di1], s1, add=True)

        @pl.when(j < NCH // 2 - 1)
        def _():
            pltpu.make_async_copy(rows0, acc_sh.at[di0], s0).wait()
            b = base0 + (i0 + 2) * C
            pltpu.sync_copy(src_h.at[pl.ds(b, C)], si0)
            pltpu.sync_copy(dst_h.at[pl.ds(b, C)], di0)
            pltpu.async_copy(y_h.at[si0], rows0, g0)
            pltpu.make_async_copy(rows1, acc_sh.at[di1], s1).wait()
            b2 = base0 + (i0 + 3) * C
            pltpu.sync_copy(src_h.at[pl.ds(b2, C)], si1)
            pltpu.sync_copy(dst_h.at[pl.ds(b2, C)], di1)
            pltpu.async_copy(y_h.at[si1], rows1, g1)

        return carry

    lax.fori_loop(jnp.int32(0), jnp.int32(NCH // 2), jbody, jnp.int32(0))
    pltpu.make_async_copy(rows0, acc_sh.at[di0], s0).wait()
    pltpu.make_async_copy(rows1, acc_sh.at[di1], s1).wait()
    plsc.subcore_barrier()
    pltpu.sync_copy(acc_sh.at[pl.ds(s * RPS, RPS)],
                    part_h.at[c, pl.ds(s * RPS, RPS)])


def _enc1_body(x_ref, dgo_ref, dgi_ref, w1_ref, y_ref, ns_ref, nd_ref):
    dgo = dgo_ref[0] + dgo_ref[1]
    dgi = dgi_ref[0] + dgi_ref[1]
    ns = jnp.where(dgo > 0, lax.rsqrt(dgo), 0.0)
    nd = jnp.where(dgi > 0, lax.rsqrt(dgi), 0.0)
    ns_ref[...] = ns
    nd_ref[...] = nd
    y_ref[...] = jnp.dot(x_ref[...] * ns, w1_ref[...],
                         preferred_element_type=f32)


def _enc1_call(x_p, dego3, degi3, W1):
    return pl.pallas_call(
        _enc1_body,
        grid=(GB,),
        in_specs=[
            pl.BlockSpec((RB, D), lambda i: (i, _Z())),
            pl.BlockSpec((2, RB, 1), lambda i: (_Z(), i, _Z())),
            pl.BlockSpec((2, RB, 1), lambda i: (_Z(), i, _Z())),
            pl.BlockSpec((D, D), lambda i: (_Z(), _Z())),
        ],
        out_specs=[
            pl.BlockSpec((RB, D), lambda i: (i, _Z())),
            pl.BlockSpec((RB, 1), lambda i: (i, _Z())),
            pl.BlockSpec((RB, 1), lambda i: (i, _Z())),
        ],
        out_shape=[jax.ShapeDtypeStruct((NP, D), f32),
                   jax.ShapeDtypeStruct((NP, 1), f32),
                   jax.ShapeDtypeStruct((NP, 1), f32)],
    )(x_p, dego3, degi3, W1)


def _enc2_body(p_ref, ns_ref, nd_ref, b1_ref, a1_ref, w2_ref, y2_ref):
    t = (p_ref[0] + p_ref[1]) * nd_ref[...] + b1_ref[...]
    h = jnp.where(t >= 0, t, a1_ref[...] * t)
    y2_ref[...] = jnp.dot(h * ns_ref[...], w2_ref[...],
                          preferred_element_type=f32)


def _enc2_call(part1, ns, nd, b1, a1, W2):
    return pl.pallas_call(
        _enc2_body,
        grid=(GB,),
        in_specs=[
            pl.BlockSpec((2, RB, D), lambda i: (_Z(), i, _Z())),
            pl.BlockSpec((RB, 1), lambda i: (i, _Z())),
            pl.BlockSpec((RB, 1), lambda i: (i, _Z())),
            pl.BlockSpec((1, D), lambda i: (_Z(), _Z())),
            pl.BlockSpec((1, 1), lambda i: (_Z(), _Z())),
            pl.BlockSpec((D, D), lambda i: (_Z(), _Z())),
        ],
        out_specs=pl.BlockSpec((RB, D), lambda i: (i, _Z())),
        out_shape=jax.ShapeDtypeStruct((NP, D), f32),
    )(part1, ns, nd, b1, a1, W2)


def _tail_body(p_ref, nd_ref, b2_ref, a2_ref, sid_ref, pres_ref, fcw_ref,
               fcb_ref, g_ref, out_ref, emb_acc, bits_s):
    i = pl.program_id(0)

    @pl.when(i < GB)
    def _():
        t = (p_ref[0] + p_ref[1]) * nd_ref[...] + b2_ref[...]
        h2 = jnp.where(t >= 0, t, a2_ref[...] * t)
        oh = (sid_ref[...] == lax.broadcasted_iota(i32, (1, SBINS), 1))
        contrib = lax.dot_general(oh.astype(f32), h2, (((0,), (0,)), ((), ())),
                                  preferred_element_type=f32)

        @pl.when(i == 0)
        def _():
            emb_acc[...] = contrib

        @pl.when(i > 0)
        def _():
            emb_acc[...] = emb_acc[...] + contrib

    @pl.when(i == GB)
    def _():
        emb = emb_acc[...]
        ind = (pres_ref[0] + pres_ref[1] > 0).astype(f32)
        row_i = lax.broadcasted_iota(i32, (SBINS, SBINS), 0)
        col_i = lax.broadcasted_iota(i32, (SBINS, SBINS), 1)
        tril = (row_i > col_i).astype(f32)
        rank = jnp.dot(tril, ind, preferred_element_type=f32)
        onehot = (rank.astype(i32) == col_i).astype(f32)
        gsel = jnp.dot(onehot, g_ref[...], preferred_element_type=f32)
        logits = lax.dot_general(emb, fcw_ref[...], (((1,), (1,)), ((), ())),
                                 preferred_element_type=f32)
        z = logits + fcb_ref[...] + gsel
        bit = (z[:, 1:2] > z[:, 0:1]).astype(f32)
        bits_s[...] = jnp.concatenate([1.0 - bit, bit], axis=1)

    @pl.when(i > GB)
    def _():
        mask = (sid_ref[...] == lax.broadcasted_iota(i32, (1, SBINS), 1))
        out_ref[...] = jnp.dot(mask.astype(f32), bits_s[...],
                               preferred_element_type=f32)


def _tail_call(part2, nd, b2, a2, sid2, pres3, fc_w, fc_b, g):
    blk1 = lambda i: jnp.minimum(i, GB - 1)
    blksid = lambda i: jnp.where(i < GB, i, jnp.maximum(i - GB - 1, 0))
    return pl.pallas_call(
        _tail_body,
        grid=(2 * GB + 1,),
        in_specs=[
            pl.BlockSpec((2, RB, D), lambda i: (_Z(), blk1(i), _Z())),
            pl.BlockSpec((RB, 1), lambda i: (blk1(i), _Z())),
            pl.BlockSpec((1, D), lambda i: (_Z(), _Z())),
            pl.BlockSpec((1, 1), lambda i: (_Z(), _Z())),
            pl.BlockSpec((RB, 1), lambda i: (blksid(i), _Z())),
            pl.BlockSpec((2, SBINS, 1), lambda i: (_Z(), _Z(), _Z())),
            pl.BlockSpec((2, D), lambda i: (_Z(), _Z())),
            pl.BlockSpec((1, 2), lambda i: (_Z(), _Z())),
            pl.BlockSpec((SBINS, 2), lambda i: (_Z(), _Z())),
        ],
        out_specs=pl.BlockSpec((RB, 2), lambda i: (jnp.maximum(i - GB - 1, 0), _Z())),
        out_shape=jax.ShapeDtypeStruct((NP, 2), f32),
        scratch_shapes=[pltpu.VMEM((SBINS, D), f32),
                        pltpu.VMEM((SBINS, 2), f32)],
    )(part2, nd, b2, a2, sid2, pres3, fc_w, fc_b, g)


def _gumbel_const():
    u = jax.random.uniform(jax.random.key(42), (NSUB, 2),
                           dtype=jnp.float64, minval=1e-10, maxval=1.0)
    g = -jnp.log(-jnp.log(u))
    return jnp.zeros((SBINS, 2), f32).at[:NSUB].set(g.astype(f32))


def kernel(x, edge_index, subgraph_id, W1, b1, a1, W2, b2, a2, fc_w, fc_b):
    W1, b1, a1 = W1.astype(f32), b1.astype(f32), a1.astype(f32)
    W2, b2, a2 = W2.astype(f32), b2.astype(f32), a2.astype(f32)
    fc_w, fc_b = fc_w.astype(f32), fc_b.astype(f32)
    src = edge_index[0].astype(i32)
    dst = edge_index[1].astype(i32)
    pad_e = N + (jnp.arange(EP - E, dtype=i32) % (NP - N))
    src_f = jnp.concatenate([src, pad_e])
    dst_f = jnp.concatenate([dst, pad_e])
    src_p = src_f.reshape(NW, NCH, C)
    dst_p = dst_f.reshape(NW, NCH, C)
    sid_p = jnp.concatenate([
        subgraph_id.astype(i32),
        NSUB + (jnp.arange(NP - N, dtype=i32) % (SBINS - NSUB))])
    sid_w = sid_p.reshape(NW, NSC, SCH)
    x_p = jnp.pad(x.astype(f32), ((0, NP - N), (0, 0)))
    zflat = jnp.zeros((NP,), f32)
    zrows = jnp.zeros((64, D), f32)

    dego_p, degi_p, pres_p = _hist_sc(src_p, dst_p, sid_w, zflat)
    y1, ns, nd = _enc1_call(x_p, dego_p.reshape(2, NP, 1),
                            degi_p.reshape(2, NP, 1), W1)
    part1 = _spmm_sc(y1, src_f, dst_f, zrows)
    y2 = _enc2_call(part1, ns, nd, b1.reshape(1, D), a1.reshape(1, 1), W2)
    part2 = _spmm_sc(y2, src_f, dst_f, zrows)
    out = _tail_call(part2, nd, b2.reshape(1, D), a2.reshape(1, 1),
                     sid_p.reshape(NP, 1), pres_p.reshape(2, SBINS, 1),
                     fc_w, fc_b.reshape(1, 2), _gumbel_const())
    return out[:N].astype(jnp.float64)


# speedup vs baseline: 1.0643x; 1.0643x over previous
import functools

import jax
import jax.numpy as jnp
from jax import lax
from jax.experimental import pallas as pl
from jax.experimental.pallas import tpu as pltpu
from jax.experimental.pallas import tpu_sc as plsc

N, E, D, NSUB = 10000, 320000, 128, 500
NP = 10240
SBINS = 512
NW = 32
C = 128
NCH = 80
EW = NCH * C
EP = NW * EW
RPS = NP // 16
SW = NP // NW
SCH = 64
NSC = SW // SCH
RB = 1024
GB = NP // RB

f32 = jnp.float32
i32 = jnp.int32
_Z = lambda: jnp.int32(0)

_mesh = plsc.VectorSubcoreMesh(core_axis_name="c", subcore_axis_name="s")


@functools.partial(
    pl.kernel,
    out_type=(jax.ShapeDtypeStruct((2, NP), f32),
              jax.ShapeDtypeStruct((2, NP), f32)),
    mesh=_mesh,
    scratch_types=[pltpu.VMEM((NCH, C), i32), pltpu.VMEM((NCH, C), i32),
                   pltpu.VMEM((C,), f32),
                   pltpu.VMEM_SHARED((NP,), f32),
                   pltpu.VMEM_SHARED((NP,), f32),
                   pltpu.SemaphoreType.DMA, pltpu.SemaphoreType.DMA,
                   pltpu.SemaphoreType.DMA, pltpu.SemaphoreType.DMA],
)
def _hist_sc(src_h, dst_h, zflat_h, dego_h, degi_h,
             sidx_all, didx_all, ones_c,
             dego_sh, degi_sh, sA, sB, sC, sD):
    c = lax.axis_index("c")
    s = lax.axis_index("s")
    w = s * 2 + c
    pltpu.sync_copy(zflat_h.at[pl.ds(0, RPS)], dego_sh.at[pl.ds(s * RPS, RPS)])
    pltpu.sync_copy(zflat_h.at[pl.ds(0, RPS)], degi_sh.at[pl.ds(s * RPS, RPS)])

    for k in range(C // 16):
        ones_c[pl.ds(k * 16, 16)] = jnp.ones((16,), f32)
    pltpu.sync_copy(src_h.at[w], sidx_all)
    pltpu.sync_copy(dst_h.at[w], didx_all)
    plsc.subcore_barrier()

    def ebody(j, carry):
        i0 = j * 2
        d0 = pltpu.async_copy(ones_c, dego_sh.at[sidx_all.at[i0]], sA, add=True)
        d1 = pltpu.async_copy(ones_c, degi_sh.at[didx_all.at[i0]], sB, add=True)
        d2 = pltpu.async_copy(ones_c, dego_sh.at[sidx_all.at[i0 + 1]], sC, add=True)
        d3 = pltpu.async_copy(ones_c, degi_sh.at[didx_all.at[i0 + 1]], sD, add=True)
        d0.wait()
        d1.wait()
        d2.wait()
        d3.wait()
        return carry

    lax.fori_loop(jnp.int32(0), jnp.int32(NCH // 2), ebody, jnp.int32(0))
    plsc.subcore_barrier()
    pltpu.sync_copy(dego_sh.at[pl.ds(s * RPS, RPS)],
                    dego_h.at[c, pl.ds(s * RPS, RPS)])
    pltpu.sync_copy(degi_sh.at[pl.ds(s * RPS, RPS)],
                    degi_h.at[c, pl.ds(s * RPS, RPS)])


@functools.partial(
    pl.kernel,
    out_type=jax.ShapeDtypeStruct((2, NP, D), f32),
    mesh=_mesh,
    scratch_types=[pltpu.VMEM((C,), i32), pltpu.VMEM((C,), i32),
                   pltpu.VMEM((C,), i32), pltpu.VMEM((C,), i32),
                   pltpu.VMEM((C, D), f32), pltpu.VMEM((C, D), f32),
                   pltpu.VMEM_SHARED((NP, D), f32),
                   pltpu.SemaphoreType.DMA, pltpu.SemaphoreType.DMA,
                   pltpu.SemaphoreType.DMA],
)
def _spmm_sc(y_h, src_h, dst_h, zrows_h, part_h,
             si0, si1, di0, di1, rows0, rows1, acc_sh, g0, g1, zs):
    c = lax.axis_index("c")
    s = lax.axis_index("s")
    w = s * 2 + c
    for k in range(RPS // 64):
        pltpu.async_copy(zrows_h.at[pl.ds(0, 64)],
                         acc_sh.at[pl.ds(s * RPS + k * 64, 64)], zs)
    for k in range(RPS // 64):
        pltpu.make_async_copy(zrows_h.at[pl.ds(0, 64)],
                              acc_sh.at[pl.ds(s * RPS + k * 64, 64)], zs).wait()
    plsc.subcore_barrier()
    base0 = w * EW
    pltpu.sync_copy(src_h.at[pl.ds(base0, C)], si0)
    pltpu.sync_copy(dst_h.at[pl.ds(base0, C)], di0)
    pltpu.async_copy(y_h.at[si0], rows0, g0)
    pltpu.sync_copy(src_h.at[pl.ds(base0 + C, C)], si1)
    pltpu.sync_copy(dst_h.at[pl.ds(base0 + C, C)], di1)
    pltpu.async_copy(y_h.at[si1], rows1, g1)

    def jbody(j, carry):
        i0 = j * 2
        pltpu.make_async_copy(y_h.at[si0], rows0, g0).wait()
        pltpu.sync_copy(rows0, acc_sh.at[di0], add=True)

        @pl.when(j < NCH // 2 - 1)
        def _():
            b = base0 + (i0 + 2) * C
            pltpu.sync_copy(src_h.at[pl.ds(b, C)], si0)
            pltpu.sync_copy(dst_h.at[pl.ds(b, C)], di0)
            pltpu.async_copy(y_h.at[si0], rows0, g0)

        pltpu.make_async_copy(y_h.at[si1], rows1, g1).wait()
        pltpu.sync_copy(rows1, acc_sh.at[di1], add=True)

        @pl.when(j < NCH // 2 - 1)
        def _():
            b2 = base0 + (i0 + 3) * C
            pltpu.sync_copy(src_h.at[pl.ds(b2, C)], si1)
            pltpu.sync_copy(dst_h.at[pl.ds(b2, C)], di1)
            pltpu.async_copy(y_h.at[si1], rows1, g1)

        return carry

    lax.fori_loop(jnp.int32(0), jnp.int32(NCH // 2), jbody, jnp.int32(0))
    plsc.subcore_barrier()
    pltpu.sync_copy(acc_sh.at[pl.ds(s * RPS, RPS)],
                    part_h.at[c, pl.ds(s * RPS, RPS)])


def _enc1_body(x_ref, dgo_ref, dgi_ref, w1_ref, y_ref, ns_ref, nd_ref):
    dgo = dgo_ref[0] + dgo_ref[1]
    dgi = dgi_ref[0] + dgi_ref[1]
    ns = jnp.where(dgo > 0, lax.rsqrt(dgo), 0.0)
    nd = jnp.where(dgi > 0, lax.rsqrt(dgi), 0.0)
    ns_ref[...] = ns
    nd_ref[...] = nd
    y_ref[...] = jnp.dot(x_ref[...] * ns, w1_ref[...],
                         preferred_element_type=f32)


def _enc1_call(x_p, dego3, degi3, W1):
    return pl.pallas_call(
        _enc1_body,
        grid=(GB,),
        in_specs=[
            pl.BlockSpec((RB, D), lambda i: (i, _Z())),
            pl.BlockSpec((2, RB, 1), lambda i: (_Z(), i, _Z())),
            pl.BlockSpec((2, RB, 1), lambda i: (_Z(), i, _Z())),
            pl.BlockSpec((D, D), lambda i: (_Z(), _Z())),
        ],
        out_specs=[
            pl.BlockSpec((RB, D), lambda i: (i, _Z())),
            pl.BlockSpec((RB, 1), lambda i: (i, _Z())),
            pl.BlockSpec((RB, 1), lambda i: (i, _Z())),
        ],
        out_shape=[jax.ShapeDtypeStruct((NP, D), f32),
                   jax.ShapeDtypeStruct((NP, 1), f32),
                   jax.ShapeDtypeStruct((NP, 1), f32)],
    )(x_p, dego3, degi3, W1)


def _enc2_body(p_ref, ns_ref, nd_ref, b1_ref, a1_ref, w2_ref, y2_ref):
    t = (p_ref[0] + p_ref[1]) * nd_ref[...] + b1_ref[...]
    h = jnp.where(t >= 0, t, a1_ref[...] * t)
    y2_ref[...] = jnp.dot(h * ns_ref[...], w2_ref[...],
                          preferred_element_type=f32)


def _enc2_call(part1, ns, nd, b1, a1, W2):
    return pl.pallas_call(
        _enc2_body,
        grid=(GB,),
        in_specs=[
            pl.BlockSpec((2, RB, D), lambda i: (_Z(), i, _Z())),
            pl.BlockSpec((RB, 1), lambda i: (i, _Z())),
            pl.BlockSpec((RB, 1), lambda i: (i, _Z())),
            pl.BlockSpec((1, D), lambda i: (_Z(), _Z())),
            pl.BlockSpec((1, 1), lambda i: (_Z(), _Z())),
            pl.BlockSpec((D, D), lambda i: (_Z(), _Z())),
        ],
        out_specs=pl.BlockSpec((RB, D), lambda i: (i, _Z())),
        out_shape=jax.ShapeDtypeStruct((NP, D), f32),
    )(part1, ns, nd, b1, a1, W2)


def _tail_body(p_ref, nd_ref, b2_ref, a2_ref, sid_ref, fcw_ref,
               fcb_ref, g_ref, out_ref, emb_acc, pres_acc, bits_s):
    i = pl.program_id(0)

    @pl.when(i < GB)
    def _():
        t = (p_ref[0] + p_ref[1]) * nd_ref[...] + b2_ref[...]
        h2 = jnp.where(t >= 0, t, a2_ref[...] * t)
        oh = (sid_ref[...] == lax.broadcasted_iota(i32, (1, SBINS), 1))
        ohf = oh.astype(f32)
        contrib = lax.dot_general(ohf, h2, (((0,), (0,)), ((), ())),
                                  preferred_element_type=f32)
        pcontrib = lax.dot_general(ohf, jnp.ones((RB, 1), f32),
                                   (((0,), (0,)), ((), ())),
                                   preferred_element_type=f32)

        @pl.when(i == 0)
        def _():
            emb_acc[...] = contrib
            pres_acc[...] = pcontrib

        @pl.when(i > 0)
        def _():
            emb_acc[...] = emb_acc[...] + contrib
            pres_acc[...] = pres_acc[...] + pcontrib

    @pl.when(i == GB)
    def _():
        emb = emb_acc[...]
        ind = (pres_acc[...] > 0).astype(f32)
        row_i = lax.broadcasted_iota(i32, (SBINS, SBINS), 0)
        col_i = lax.broadcasted_iota(i32, (SBINS, SBINS), 1)
        tril = (row_i > col_i).astype(f32)
        rank = jnp.dot(tril, ind, preferred_element_type=f32)
        onehot = (rank.astype(i32) == col_i).astype(f32)
        gsel = jnp.dot(onehot, g_ref[...], preferred_element_type=f32)
        logits = lax.dot_general(emb, fcw_ref[...], (((1,), (1,)), ((), ())),
                                 preferred_element_type=f32)
        z = logits + fcb_ref[...] + gsel
        bit = (z[:, 1:2] > z[:, 0:1]).astype(f32)
        bits_s[...] = jnp.concatenate([1.0 - bit, bit], axis=1)

    @pl.when(i > GB)
    def _():
        mask = (sid_ref[...] == lax.broadcasted_iota(i32, (1, SBINS), 1))
        out_ref[...] = jnp.dot(mask.astype(f32), bits_s[...],
                               preferred_element_type=f32)


def _tail_call(part2, nd, b2, a2, sid2, fc_w, fc_b, g):
    blk1 = lambda i: jnp.minimum(i, GB - 1)
    blksid = lambda i: jnp.where(i < GB, i, jnp.maximum(i - GB - 1, 0))
    return pl.pallas_call(
        _tail_body,
        grid=(2 * GB + 1,),
        in_specs=[
            pl.BlockSpec((2, RB, D), lambda i: (_Z(), blk1(i), _Z())),
            pl.BlockSpec((RB, 1), lambda i: (blk1(i), _Z())),
            pl.BlockSpec((1, D), lambda i: (_Z(), _Z())),
            pl.BlockSpec((1, 1), lambda i: (_Z(), _Z())),
            pl.BlockSpec((RB, 1), lambda i: (blksid(i), _Z())),
            pl.BlockSpec((2, D), lambda i: (_Z(), _Z())),
            pl.BlockSpec((1, 2), lambda i: (_Z(), _Z())),
            pl.BlockSpec((SBINS, 2), lambda i: (_Z(), _Z())),
        ],
        out_specs=pl.BlockSpec((RB, 2), lambda i: (jnp.maximum(i - GB - 1, 0), _Z())),
        out_shape=jax.ShapeDtypeStruct((NP, 2), f32),
        scratch_shapes=[pltpu.VMEM((SBINS, D), f32),
                        pltpu.VMEM((SBINS, 1), f32),
                        pltpu.VMEM((SBINS, 2), f32)],
    )(part2, nd, b2, a2, sid2, fc_w, fc_b, g)


def _gumbel_const():
    u = jax.random.uniform(jax.random.key(42), (NSUB, 2),
                           dtype=jnp.float64, minval=1e-10, maxval=1.0)
    g = -jnp.log(-jnp.log(u))
    return jnp.zeros((SBINS, 2), f32).at[:NSUB].set(g.astype(f32))


def kernel(x, edge_index, subgraph_id, W1, b1, a1, W2, b2, a2, fc_w, fc_b):
    W1, b1, a1 = W1.astype(f32), b1.astype(f32), a1.astype(f32)
    W2, b2, a2 = W2.astype(f32), b2.astype(f32), a2.astype(f32)
    fc_w, fc_b = fc_w.astype(f32), fc_b.astype(f32)
    src = edge_index[0].astype(i32)
    dst = edge_index[1].astype(i32)
    pad_e = N + (jnp.arange(EP - E, dtype=i32) % (NP - N))
    src_f = jnp.concatenate([src, pad_e])
    dst_f = jnp.concatenate([dst, pad_e])
    src_p = src_f.reshape(NW, NCH, C)
    dst_p = dst_f.reshape(NW, NCH, C)
    sid_p = jnp.concatenate([
        subgraph_id.astype(i32),
        NSUB + (jnp.arange(NP - N, dtype=i32) % (SBINS - NSUB))])
    x_p = jnp.pad(x.astype(f32), ((0, NP - N), (0, 0)))
    zflat = jnp.zeros((NP,), f32)
    zrows = jnp.zeros((64, D), f32)

    dego_p, degi_p = _hist_sc(src_p, dst_p, zflat)
    y1, ns, nd = _enc1_call(x_p, dego_p.reshape(2, NP, 1),
                            degi_p.reshape(2, NP, 1), W1)
    part1 = _spmm_sc(y1, src_f, dst_f, zrows)
    y2 = _enc2_call(part1, ns, nd, b1.reshape(1, D), a1.reshape(1, 1), W2)
    part2 = _spmm_sc(y2, src_f, dst_f, zrows)
    out = _tail_call(part2, nd, b2.reshape(1, D), a2.reshape(1, 1),
                     sid_p.reshape(NP, 1),
                     fc_w, fc_b.reshape(1, 2), _gumbel_const())
    return out[:N].astype(jnp.float64)


# speedup vs baseline: 1.0656x; 1.0013x over previous
import functools

import jax
import jax.numpy as jnp
from jax import lax
from jax.experimental import pallas as pl
from jax.experimental.pallas import tpu as pltpu
from jax.experimental.pallas import tpu_sc as plsc

N, E, D, NSUB = 10000, 320000, 128, 500
NP = 10240
SBINS = 512
NW = 32
C = 128
NCH = 80
EW = NCH * C
EP = NW * EW
RPS = NP // 16
RB = 1024
GB = NP // RB

f32 = jnp.float32
i32 = jnp.int32
_Z = lambda: jnp.int32(0)

_mesh = plsc.VectorSubcoreMesh(core_axis_name="c", subcore_axis_name="s")


@functools.partial(
    pl.kernel,
    out_type=(jax.ShapeDtypeStruct((2, NP), f32),
              jax.ShapeDtypeStruct((2, NP), f32)),
    mesh=_mesh,
    scratch_types=[pltpu.VMEM((NCH, C), i32), pltpu.VMEM((NCH, C), i32),
                   pltpu.VMEM((C,), f32),
                   pltpu.VMEM_SHARED((NP,), f32),
                   pltpu.VMEM_SHARED((NP,), f32),
                   pltpu.SemaphoreType.DMA, pltpu.SemaphoreType.DMA],
)
def _hist_sc(src_h, dst_h, zflat_h, dego_h, degi_h,
             sidx_all, didx_all, ones_c,
             dego_sh, degi_sh, sA, sB):
    c = lax.axis_index("c")
    s = lax.axis_index("s")
    w = s * 2 + c
    pltpu.sync_copy(zflat_h.at[pl.ds(0, RPS)], dego_sh.at[pl.ds(s * RPS, RPS)])
    pltpu.sync_copy(zflat_h.at[pl.ds(0, RPS)], degi_sh.at[pl.ds(s * RPS, RPS)])

    for k in range(C // 16):
        ones_c[pl.ds(k * 16, 16)] = jnp.ones((16,), f32)
    pltpu.sync_copy(src_h.at[w], sidx_all)
    pltpu.sync_copy(dst_h.at[w], didx_all)
    plsc.subcore_barrier()

    def ebody(j, carry):
        i0 = j * 4
        ds_ = []
        for k in range(4):
            ds_.append(pltpu.async_copy(
                ones_c, dego_sh.at[sidx_all.at[i0 + k]], sA, add=True))
            ds_.append(pltpu.async_copy(
                ones_c, degi_sh.at[didx_all.at[i0 + k]], sB, add=True))
        for d in ds_:
            d.wait()
        return carry

    lax.fori_loop(jnp.int32(0), jnp.int32(NCH // 4), ebody, jnp.int32(0))
    plsc.subcore_barrier()
    pltpu.sync_copy(dego_sh.at[pl.ds(s * RPS, RPS)],
                    dego_h.at[c, pl.ds(s * RPS, RPS)])
    pltpu.sync_copy(degi_sh.at[pl.ds(s * RPS, RPS)],
                    degi_h.at[c, pl.ds(s * RPS, RPS)])


@functools.partial(
    pl.kernel,
    out_type=jax.ShapeDtypeStruct((2, NP, D), f32),
    mesh=_mesh,
    scratch_types=[pltpu.VMEM((C,), i32), pltpu.VMEM((C,), i32),
                   pltpu.VMEM((C,), i32), pltpu.VMEM((C,), i32),
                   pltpu.VMEM((C, D), f32), pltpu.VMEM((C, D), f32),
                   pltpu.VMEM_SHARED((NP, D), f32),
                   pltpu.SemaphoreType.DMA, pltpu.SemaphoreType.DMA,
                   pltpu.SemaphoreType.DMA],
)
def _spmm_sc(y_h, src_h, dst_h, zrows_h, part_h,
             si0, si1, di0, di1, rows0, rows1, acc_sh, g0, g1, zs):
    c = lax.axis_index("c")
    s = lax.axis_index("s")
    w = s * 2 + c
    for k in range(RPS // 64):
        pltpu.async_copy(zrows_h.at[pl.ds(0, 64)],
                         acc_sh.at[pl.ds(s * RPS + k * 64, 64)], zs)
    for k in range(RPS // 64):
        pltpu.make_async_copy(zrows_h.at[pl.ds(0, 64)],
                              acc_sh.at[pl.ds(s * RPS + k * 64, 64)], zs).wait()
    plsc.subcore_barrier()
    base0 = w * EW
    pltpu.sync_copy(src_h.at[pl.ds(base0, C)], si0)
    pltpu.sync_copy(dst_h.at[pl.ds(base0, C)], di0)
    pltpu.async_copy(y_h.at[si0], rows0, g0)
    pltpu.sync_copy(src_h.at[pl.ds(base0 + C, C)], si1)
    pltpu.sync_copy(dst_h.at[pl.ds(base0 + C, C)], di1)
    pltpu.async_copy(y_h.at[si1], rows1, g1)

    def jbody(j, carry):
        i0 = j * 2
        pltpu.make_async_copy(y_h.at[si0], rows0, g0).wait()
        pltpu.sync_copy(rows0, acc_sh.at[di0], add=True)

        @pl.when(j < NCH // 2 - 1)
        def _():
            b = base0 + (i0 + 2) * C
            pltpu.sync_copy(src_h.at[pl.ds(b, C)], si0)
            pltpu.sync_copy(dst_h.at[pl.ds(b, C)], di0)
            pltpu.async_copy(y_h.at[si0], rows0, g0)

        pltpu.make_async_copy(y_h.at[si1], rows1, g1).wait()
        pltpu.sync_copy(rows1, acc_sh.at[di1], add=True)

        @pl.when(j < NCH // 2 - 1)
        def _():
            b2 = base0 + (i0 + 3) * C
            pltpu.sync_copy(src_h.at[pl.ds(b2, C)], si1)
            pltpu.sync_copy(dst_h.at[pl.ds(b2, C)], di1)
            pltpu.async_copy(y_h.at[si1], rows1, g1)

        return carry

    lax.fori_loop(jnp.int32(0), jnp.int32(NCH // 2), jbody, jnp.int32(0))
    plsc.subcore_barrier()
    pltpu.sync_copy(acc_sh.at[pl.ds(s * RPS, RPS)],
                    part_h.at[c, pl.ds(s * RPS, RPS)])


def _enc1_body(x_ref, dgo_ref, dgi_ref, w1_ref, y_ref, ns_ref, nd_ref):
    dgo = dgo_ref[0] + dgo_ref[1]
    dgi = dgi_ref[0] + dgi_ref[1]
    ns = jnp.where(dgo > 0, lax.rsqrt(dgo), 0.0)
    nd = jnp.where(dgi > 0, lax.rsqrt(dgi), 0.0)
    ns_ref[...] = ns
    nd_ref[...] = nd
    y_ref[...] = jnp.dot(x_ref[...] * ns, w1_ref[...],
                         preferred_element_type=f32)


def _enc1_call(x_p, dego3, degi3, W1):
    return pl.pallas_call(
        _enc1_body,
        grid=(GB,),
        in_specs=[
            pl.BlockSpec((RB, D), lambda i: (i, _Z())),
            pl.BlockSpec((2, RB, 1), lambda i: (_Z(), i, _Z())),
            pl.BlockSpec((2, RB, 1), lambda i: (_Z(), i, _Z())),
            pl.BlockSpec((D, D), lambda i: (_Z(), _Z())),
        ],
        out_specs=[
            pl.BlockSpec((RB, D), lambda i: (i, _Z())),
            pl.BlockSpec((RB, 1), lambda i: (i, _Z())),
            pl.BlockSpec((RB, 1), lambda i: (i, _Z())),
        ],
        out_shape=[jax.ShapeDtypeStruct((NP, D), f32),
                   jax.ShapeDtypeStruct((NP, 1), f32),
                   jax.ShapeDtypeStruct((NP, 1), f32)],
    )(x_p, dego3, degi3, W1)


def _enc2_body(p_ref, ns_ref, nd_ref, b1_ref, a1_ref, w2_ref, y2_ref):
    t = (p_ref[0] + p_ref[1]) * nd_ref[...] + b1_ref[...]
    h = jnp.where(t >= 0, t, a1_ref[...] * t)
    y2_ref[...] = jnp.dot(h * ns_ref[...], w2_ref[...],
                          preferred_element_type=f32)


def _enc2_call(part1, ns, nd, b1, a1, W2):
    return pl.pallas_call(
        _enc2_body,
        grid=(GB,),
        in_specs=[
            pl.BlockSpec((2, RB, D), lambda i: (_Z(), i, _Z())),
            pl.BlockSpec((RB, 1), lambda i: (i, _Z())),
            pl.BlockSpec((RB, 1), lambda i: (i, _Z())),
            pl.BlockSpec((1, D), lambda i: (_Z(), _Z())),
            pl.BlockSpec((1, 1), lambda i: (_Z(), _Z())),
            pl.BlockSpec((D, D), lambda i: (_Z(), _Z())),
        ],
        out_specs=pl.BlockSpec((RB, D), lambda i: (i, _Z())),
        out_shape=jax.ShapeDtypeStruct((NP, D), f32),
    )(part1, ns, nd, b1, a1, W2)


def _tail_body(p_ref, nd_ref, b2_ref, a2_ref, sid_ref, fcw_ref,
               fcb_ref, g_ref, out_ref, emb_acc, pres_acc, bits_s):
    i = pl.program_id(0)

    @pl.when(i < GB)
    def _():
        t = (p_ref[0] + p_ref[1]) * nd_ref[...] + b2_ref[...]
        h2 = jnp.where(t >= 0, t, a2_ref[...] * t)
        oh = (sid_ref[...] == lax.broadcasted_iota(i32, (1, SBINS), 1))
        ohf = oh.astype(f32)
        contrib = lax.dot_general(ohf, h2, (((0,), (0,)), ((), ())),
                                  preferred_element_type=f32)
        pcontrib = lax.dot_general(ohf, jnp.ones((RB, 1), f32),
                                   (((0,), (0,)), ((), ())),
                                   preferred_element_type=f32)

        @pl.when(i == 0)
        def _():
            emb_acc[...] = contrib
            pres_acc[...] = pcontrib

        @pl.when(i > 0)
        def _():
            emb_acc[...] = emb_acc[...] + contrib
            pres_acc[...] = pres_acc[...] + pcontrib

    @pl.when(i == GB)
    def _():
        emb = emb_acc[...]
        ind = (pres_acc[...] > 0).astype(f32)
        row_i = lax.broadcasted_iota(i32, (SBINS, SBINS), 0)
        col_i = lax.broadcasted_iota(i32, (SBINS, SBINS), 1)
        tril = (row_i > col_i).astype(f32)
        rank = jnp.dot(tril, ind, preferred_element_type=f32)
        onehot = (rank.astype(i32) == col_i).astype(f32)
        gsel = jnp.dot(onehot, g_ref[...], preferred_element_type=f32)
        logits = lax.dot_general(emb, fcw_ref[...], (((1,), (1,)), ((), ())),
                                 preferred_element_type=f32)
        z = logits + fcb_ref[...] + gsel
        bit = (z[:, 1:2] > z[:, 0:1]).astype(f32)
        bits_s[...] = jnp.concatenate([1.0 - bit, bit], axis=1)

    @pl.when(i > GB)
    def _():
        mask = (sid_ref[...] == lax.broadcasted_iota(i32, (1, SBINS), 1))
        out_ref[...] = jnp.dot(mask.astype(f32), bits_s[...],
                               preferred_element_type=f32)


def _tail_call(part2, nd, b2, a2, sid2, fc_w, fc_b, g):
    blk1 = lambda i: jnp.minimum(i, GB - 1)
    blksid = lambda i: jnp.where(i < GB, i, jnp.maximum(i - GB - 1, 0))
    return pl.pallas_call(
        _tail_body,
        grid=(2 * GB + 1,),
        in_specs=[
            pl.BlockSpec((2, RB, D), lambda i: (_Z(), blk1(i), _Z())),
            pl.BlockSpec((RB, 1), lambda i: (blk1(i), _Z())),
            pl.BlockSpec((1, D), lambda i: (_Z(), _Z())),
            pl.BlockSpec((1, 1), lambda i: (_Z(), _Z())),
            pl.BlockSpec((RB, 1), lambda i: (blksid(i), _Z())),
            pl.BlockSpec((2, D), lambda i: (_Z(), _Z())),
            pl.BlockSpec((1, 2), lambda i: (_Z(), _Z())),
            pl.BlockSpec((SBINS, 2), lambda i: (_Z(), _Z())),
        ],
        out_specs=pl.BlockSpec((RB, 2), lambda i: (jnp.maximum(i - GB - 1, 0), _Z())),
        out_shape=jax.ShapeDtypeStruct((NP, 2), f32),
        scratch_shapes=[pltpu.VMEM((SBINS, D), f32),
                        pltpu.VMEM((SBINS, 1), f32),
                        pltpu.VMEM((SBINS, 2), f32)],
    )(part2, nd, b2, a2, sid2, fc_w, fc_b, g)


def _gumbel_const():
    u = jax.random.uniform(jax.random.key(42), (NSUB, 2),
                           dtype=jnp.float64, minval=1e-10, maxval=1.0)
    g = -jnp.log(-jnp.log(u))
    return jnp.zeros((SBINS, 2), f32).at[:NSUB].set(g.astype(f32))


def kernel(x, edge_index, subgraph_id, W1, b1, a1, W2, b2, a2, fc_w, fc_b):
    W1, b1, a1 = W1.astype(f32), b1.astype(f32), a1.astype(f32)
    W2, b2, a2 = W2.astype(f32), b2.astype(f32), a2.astype(f32)
    fc_w, fc_b = fc_w.astype(f32), fc_b.astype(f32)
    src = edge_index[0].astype(i32)
    dst = edge_index[1].astype(i32)
    pad_e = N + (jnp.arange(EP - E, dtype=i32) % (NP - N))
    src_f = jnp.concatenate([src, pad_e])
    dst_f = jnp.concatenate([dst, pad_e])
    src_p = src_f.reshape(NW, NCH, C)
    dst_p = dst_f.reshape(NW, NCH, C)
    sid_p = jnp.concatenate([
        subgraph_id.astype(i32),
        NSUB + (jnp.arange(NP - N, dtype=i32) % (SBINS - NSUB))])
    x_p = jnp.pad(x.astype(f32), ((0, NP - N), (0, 0)))
    zflat = jnp.zeros((NP,), f32)
    zrows = jnp.zeros((64, D), f32)

    dego_p, degi_p = _hist_sc(src_p, dst_p, zflat)
    y1, ns, nd = _enc1_call(x_p, dego_p.reshape(2, NP, 1),
                            degi_p.reshape(2, NP, 1), W1)
    part1 = _spmm_sc(y1, src_f, dst_f, zrows)
    y2 = _enc2_call(part1, ns, nd, b1.reshape(1, D), a1.reshape(1, 1), W2)
    part2 = _spmm_sc(y2, src_f, dst_f, zrows)
    out = _tail_call(part2, nd, b2.reshape(1, D), a2.reshape(1, 1),
                     sid_p.reshape(NP, 1),
                     fc_w, fc_b.reshape(1, 2), _gumbel_const())
    return out[:N].astype(jnp.float64)
